# baseline (speedup 1.0000x reference)
import functools

import jax
import jax.numpy as jnp
from jax import lax
from jax.experimental import pallas as pl
from jax.experimental.pallas import tpu as pltpu
from jax.experimental.pallas import tpu_sc as plsc

D_MODEL = 64
BATCH = 1024
SEQ_LEN = 200
NC, NS, LANES = 2, 16, 16
NW = NC * NS
ROWS = BATCH * SEQ_LEN
RPW = ROWS // NW
CHUNK = 100
NCH = RPW // CHUNK


def _pos_encoding(seq_len, d_model):
    i_model = jnp.repeat(jnp.arange(d_model // 2), 2)
    div_term = jnp.exp(
        i_model.astype(jnp.float32) / d_model * jnp.log(jnp.float32(10000.0))
    )
    pos = jnp.arange(seq_len, dtype=jnp.float32)[:, None] / div_term
    even = (jnp.arange(d_model) % 2) == 0
    return jnp.where(even[None, :], jnp.sin(pos), jnp.cos(pos))


def _body(x_ref, tab_ref, pe_ref, out_ref, idx_v, buf, pe_v, gsem):
    wid = lax.axis_index("s") * NC + lax.axis_index("c")
    pltpu.sync_copy(x_ref.at[wid], idx_v)
    pltpu.sync_copy(pe_ref, pe_v)

    @pl.loop(0, NCH // 2)
    def _pair(t):
        for p in range(2):
            c = t * 2 + p
            pltpu.async_copy(tab_ref.at[idx_v.at[c]], buf, gsem).wait()
            pev = pe_v.at[p]

            @pl.loop(0, CHUNK)
            def _row(r):
                for k in range(D_MODEL // LANES):
                    sl = pl.ds(k * LANES, LANES)
                    buf[r, sl] = buf[r, sl] + pev[r, sl]

            pltpu.sync_copy(buf, out_ref.at[wid * NCH + c])


@functools.partial(jax.jit, static_argnums=())
def _emb_lookup(x3, emb_weight, pe3):
    mesh = plsc.VectorSubcoreMesh(
        core_axis_name="c", subcore_axis_name="s", num_cores=NC, num_subcores=NS
    )
    f = pl.kernel(
        _body,
        out_type=jax.ShapeDtypeStruct((NW * NCH, CHUNK, D_MODEL), jnp.float32),
        mesh=mesh,
        scratch_types=[
            pltpu.VMEM((NCH, CHUNK), jnp.int32),
            pltpu.VMEM((CHUNK, D_MODEL), jnp.float32),
            pltpu.VMEM((2, CHUNK, D_MODEL), jnp.float32),
            pltpu.SemaphoreType.DMA,
        ],
        compiler_params=pltpu.CompilerParams(use_tc_tiling_on_sc=False),
    )
    return f(x3, emb_weight, pe3)


def kernel(x, emb_weight):
    pe3 = _pos_encoding(SEQ_LEN, D_MODEL).reshape(2, CHUNK, D_MODEL)
    x3 = x.reshape(NW, NCH, CHUNK)
    out = _emb_lookup(x3, emb_weight, pe3)
    return out.reshape(BATCH, SEQ_LEN, D_MODEL)


# speedup vs baseline: 1.0268x; 1.0268x over previous
import functools

import jax
import jax.numpy as jnp
from jax import lax
from jax.experimental import pallas as pl
from jax.experimental.pallas import tpu as pltpu
from jax.experimental.pallas import tpu_sc as plsc

D_MODEL = 64
BATCH = 1024
SEQ_LEN = 200
NC, NS, LANES = 2, 16, 16
NW = NC * NS
ROWS = BATCH * SEQ_LEN
RPW = ROWS // NW
CHUNK = 100
NCH = RPW // CHUNK


def _pos_encoding(seq_len, d_model):
    i_model = jnp.repeat(jnp.arange(d_model // 2), 2)
    div_term = jnp.exp(
        i_model.astype(jnp.float32) / d_model * jnp.log(jnp.float32(10000.0))
    )
    pos = jnp.arange(seq_len, dtype=jnp.float32)[:, None] / div_term
    even = (jnp.arange(d_model) % 2) == 0
    return jnp.where(even[None, :], jnp.sin(pos), jnp.cos(pos))


NBUF = 8


def _body(x_ref, tab_ref, pe_ref, out_ref, idx_v, bufs, pe_v, *sems):
    gsems, osems = sems[:NBUF], sems[NBUF:]
    wid = lax.axis_index("s") * NC + lax.axis_index("c")
    pltpu.sync_copy(x_ref.at[wid], idx_v)
    pltpu.sync_copy(pe_ref, pe_v)

    @pl.loop(0, NCH // NBUF)
    def _group(t):
        c0 = t * NBUF
        gds = [
            pltpu.async_copy(tab_ref.at[idx_v.at[c0 + b]], bufs.at[b], gsems[b])
            for b in range(NBUF)
        ]
        ods = []
        for b in range(NBUF):
            gds[b].wait()
            buf = bufs.at[b]
            pev = pe_v.at[b % 2]

            @pl.loop(0, CHUNK, unroll=10)
            def _row(r):
                for k in range(D_MODEL // LANES):
                    sl = pl.ds(k * LANES, LANES)
                    buf[r, sl] = buf[r, sl] + pev[r, sl]

            ods.append(
                pltpu.async_copy(buf, out_ref.at[wid * NCH + c0 + b], osems[b])
            )
        for d in ods:
            d.wait()


@functools.partial(jax.jit, static_argnums=())
def _emb_lookup(x3, emb_weight, pe3):
    mesh = plsc.VectorSubcoreMesh(
        core_axis_name="c", subcore_axis_name="s", num_cores=NC, num_subcores=NS
    )
    f = pl.kernel(
        _body,
        out_type=jax.ShapeDtypeStruct((NW * NCH, CHUNK, D_MODEL), jnp.float32),
        mesh=mesh,
        scratch_types=[
            pltpu.VMEM((NCH, CHUNK), jnp.int32),
            pltpu.VMEM((NBUF, CHUNK, D_MODEL), jnp.float32),
            pltpu.VMEM((2, CHUNK, D_MODEL), jnp.float32),
        ]
        + [pltpu.SemaphoreType.DMA] * (2 * NBUF),
        compiler_params=pltpu.CompilerParams(use_tc_tiling_on_sc=False),
    )
    return f(x3, emb_weight, pe3)


def kernel(x, emb_weight):
    pe3 = _pos_encoding(SEQ_LEN, D_MODEL).reshape(2, CHUNK, D_MODEL)
    x3 = x.reshape(NW, NCH, CHUNK)
    out = _emb_lookup(x3, emb_weight, pe3)
    return out.reshape(BATCH, SEQ_LEN, D_MODEL)
